# baseline (speedup 1.0000x reference)
import functools

import numpy as np
import jax
import jax.numpy as jnp
from jax import lax
from jax.experimental import pallas as pl
from jax.experimental.pallas import tpu as pltpu
from jax.experimental.pallas import tpu_sc as plsc

_D = 64
_BATCH = 4096
_SEQ = 200
_COND = 128
_MAXVAL = 100.0

_NC = 2
_NS = 16
_NW = _NC * _NS
_BB = 512
_SG = _SEQ // 8


def _tc_body(t_ref, cond_ref, w_ref, te_ref, co_ref):
    t = t_ref[:]
    row = lax.broadcasted_iota(jnp.int32, (_D, _BB), 0)
    half = jnp.where(row < _D // 2, row, row - _D // 2).astype(jnp.float32)
    inv_freq = jnp.exp(half * (-2.0 * float(np.log(_MAXVAL)) / _D))
    arg = t * inv_freq
    te_ref[:] = jnp.where(row < _D // 2, jnp.sin(arg), jnp.cos(arg))
    co_ref[:] = lax.dot_general(
        w_ref[:], cond_ref[:],
        dimension_numbers=(((1,), (1,)), ((), ())),
        preferred_element_type=jnp.float32,
    )


_tc_call = pl.pallas_call(
    _tc_body,
    grid=(_BATCH // _BB,),
    in_specs=[
        pl.BlockSpec((1, _BB), lambda i: (0, i)),
        pl.BlockSpec((_BB, _COND), lambda i: (i, 0)),
        pl.BlockSpec((_D, _COND), lambda i: (0, 0)),
    ],
    out_specs=[
        pl.BlockSpec((_D, _BB), lambda i: (0, i)),
        pl.BlockSpec((_D, _BB), lambda i: (0, i)),
    ],
    out_shape=[
        jax.ShapeDtypeStruct((_D, _BATCH), jnp.float32),
        jax.ShapeDtypeStruct((_D, _BATCH), jnp.float32),
    ],
)


def _sc_body(x_ref, te_ref, tab_ref, out_ref,
             idx_v, rows0, rows1, blk0, blk1, gsem0, gsem1, wsem):
    w = lax.axis_index("s") * _NC + lax.axis_index("c")
    b0 = w * 128

    rows = (rows0, rows1)
    gsem = (gsem0, gsem1)
    blk = (blk0, blk1)

    pltpu.sync_copy(x_ref.at[:, pl.ds(b0, 128)], idx_v)

    pltpu.sync_copy(te_ref.at[:, :, :, :, pl.ds(b0, 128)], blk0)
    pltpu.sync_copy(blk0, out_ref.at[pl.ds(0, 1), :, pl.ds(w, 1)])

    ridx = [jnp.arange(16, dtype=jnp.int32) + 16 * bg for bg in range(8)]

    def fire_gather(s, p):
        pltpu.async_copy(tab_ref.at[idx_v.at[s - 1]],
                         rows[p], gsem[p])

    def gwait(p):
        pltpu.make_async_copy(tab_ref.at[pl.ds(0, 128)], rows[p],
                              gsem[p]).wait()

    def wwait():
        pltpu.make_async_copy(blk0, out_ref.at[pl.ds(0, 1), :, pl.ds(0, 1)],
                              wsem).wait()

    def transpose_store(s, p):
        for dI in range(8):
            for dlo in range(8):
                cidx = jnp.full((16,), 8 * dI + dlo, jnp.int32)
                for bg in range(8):
                    v = plsc.load_gather(rows[p], [ridx[bg], cidx])
                    blk[p][0, dI, 0, dlo, pl.ds(16 * bg, 16)] = v
        pltpu.async_copy(blk[p], out_ref.at[pl.ds(s, 1), :, pl.ds(w, 1)], wsem)

    fire_gather(1, 0)
    fire_gather(2, 1)

    def pair(k, c):
        s_a = 2 * k + 1

        @pl.when(k >= 1)
        def _():
            wwait()
        gwait(0)
        transpose_store(s_a, 0)

        @pl.when(k <= _SEQ // 2 - 2)
        def _():
            fire_gather(s_a + 2, 0)

        @pl.when(k >= 1)
        def _():
            wwait()
        gwait(1)
        transpose_store(s_a + 1, 1)

        @pl.when(k <= _SEQ // 2 - 2)
        def _():
            fire_gather(s_a + 3, 1)
        return c

    lax.fori_loop(0, _SEQ // 2, pair, 0)
    wwait()
    wwait()


_sc_embed = pl.kernel(
    _sc_body,
    out_type=jax.ShapeDtypeStruct((_SEQ + 1, 8, _NW, 8, 128), jnp.float32),
    mesh=plsc.VectorSubcoreMesh(core_axis_name="c", subcore_axis_name="s"),
    scratch_types=[
        pltpu.VMEM((_SEQ, 128), jnp.int32),
        pltpu.VMEM((128, _D), jnp.float32),
        pltpu.VMEM((128, _D), jnp.float32),
        pltpu.VMEM((1, 8, 1, 8, 128), jnp.float32),
        pltpu.VMEM((1, 8, 1, 8, 128), jnp.float32),
        pltpu.SemaphoreType.DMA,
        pltpu.SemaphoreType.DMA,
        pltpu.SemaphoreType.DMA,
    ],
    compiler_params=pltpu.CompilerParams(use_tc_tiling_on_sc=False,
                                         needs_layout_passes=False),
)


@jax.jit
def kernel(x, t, condition_emb, x_emb_table, cond_weight):
    teT, coT = _tc_call(t.reshape(1, _BATCH), condition_emb, cond_weight)
    out5 = _sc_embed(x.T, teT.reshape(1, 8, 1, 8, _BATCH), x_emb_table)
    out = out5.transpose(2, 4, 0, 1, 3).reshape(_BATCH, _SEQ + 1, _D)
    return out, coT.T
